# baseline (speedup 1.0000x reference)
import functools

import jax
import jax.numpy as jnp
from jax import lax
from jax.experimental import pallas as pl
from jax.experimental.pallas import tpu as pltpu
from jax.experimental.pallas import tpu_sc as plsc

VOCAB = 1000000
DIM = 64
B = 4096
L = 200
NLAB = 1000

NC = 2
NS = 16
NW = NC * NS
DOCS_PER_TILE = B // NW
TOK_PER_TILE = DOCS_PER_TILE * L
WDIM = 2 * DIM

CHW = 128
NCHUNK = TOK_PER_TILE // CHW
NBUF = 4
DIV_MUL = 5243
DIV_SHIFT = 20


def _sc_segsum(doc_flat, table):
    mesh = plsc.VectorSubcoreMesh(core_axis_name="c", subcore_axis_name="s")

    @functools.partial(
        pl.kernel,
        mesh=mesh,
        out_type=jax.ShapeDtypeStruct((B, DIM), jnp.float32),
        compiler_params=pltpu.CompilerParams(use_tc_tiling_on_sc=False),
        scratch_types=[
            pltpu.VMEM((TOK_PER_TILE + 16,), jnp.int32),
            pltpu.VMEM((NCHUNK, CHW), jnp.int32),
            pltpu.VMEM((NBUF, CHW, DIM), jnp.float32),
            pltpu.VMEM_SHARED((NS * DOCS_PER_TILE, DIM), jnp.float32),
        ] + [pltpu.SemaphoreType.DMA] * (2 * NBUF),
    )
    def segsum(doc_hbm, tab_hbm, out_hbm, idx_v, didx_v, rows_v, acc_v,
               *sems):
        gsems = sems[:NBUF]
        ssems = sems[NBUF:]
        srow = lax.axis_index("s") * DOCS_PER_TILE
        wid = lax.axis_index("s") * NC + lax.axis_index("c")
        base = wid * TOK_PER_TILE

        pltpu.sync_copy(doc_hbm.at[pl.ds(base, TOK_PER_TILE)],
                        idx_v.at[pl.ds(0, TOK_PER_TILE)])

        lanes = lax.iota(jnp.int32, 16)

        def fill_didx(c, _):
            for j in range(CHW // 16):
                t0 = c * CHW + 16 * j
                didx_v[c, pl.ds(16 * j, 16)] = srow + jnp.right_shift(
                    (t0 + lanes) * DIV_MUL, DIV_SHIFT)
            return _

        lax.fori_loop(0, NCHUNK, fill_didx, 0)

        zero = jnp.zeros((16,), jnp.float32)

        def zero_stage(r, _):
            for d in range(DIM // 16):
                rows_v[0, r, pl.ds(16 * d, 16)] = zero
            return _

        lax.fori_loop(0, CHW, zero_stage, 0)
        pltpu.sync_copy(rows_v.at[0],
                        acc_v.at[pl.ds(srow, DOCS_PER_TILE)])

        def gather(c, buf):
            return pltpu.make_async_copy(
                tab_hbm.at[idx_v.at[pl.ds(c * CHW, CHW)]],
                rows_v.at[buf], gsems[buf])

        def scatter(c, buf):
            return pltpu.make_async_copy(
                rows_v.at[buf], acc_v.at[didx_v.at[c]], ssems[buf])

        gather(0, 0).start()

        def chunk_group(c4, _):
            for phase in range(NBUF):
                c = NBUF * c4 + phase
                buf = phase
                nbuf = (phase + 1) % NBUF

                @pl.when(jnp.logical_and(c + 1 < NCHUNK, c >= NBUF - 1))
                def _reclaim():
                    scatter(c, nbuf).wait()

                @pl.when(c + 1 < NCHUNK)
                def _prefetch():
                    gather(c + 1, nbuf).start()

                gather(c, buf).wait()
                scatter(c, buf).start(add=True)
            return _

        lax.fori_loop(0, NCHUNK // NBUF, chunk_group, 0)
        for buf in range(NBUF):
            scatter(0, buf).wait()

        pltpu.sync_copy(
            acc_v.at[pl.ds(srow, DOCS_PER_TILE)],
            out_hbm.at[pl.ds(wid * DOCS_PER_TILE, DOCS_PER_TILE)])

    return segsum(doc_flat, table)


def _tc_body(acc_ref, doc_ref, row0_ref, wd_ref, bd_ref, out_ref):
    npad = jnp.sum((doc_ref[...] == 0).astype(jnp.float32), axis=1,
                   keepdims=True)
    cnt = jnp.maximum(float(L) - npad, 1.0)
    enc = (acc_ref[...] - npad * row0_ref[...]) / cnt
    out_ref[...] = jnp.dot(enc, wd_ref[...],
                           preferred_element_type=jnp.float32) + bd_ref[...]


def _tc_decode(acc, doc, row0, Wd, bd2):
    bm = 512
    grid = B // bm
    return pl.pallas_call(
        _tc_body,
        grid=(grid,),
        in_specs=[
            pl.BlockSpec((bm, DIM), lambda i: (i, 0)),
            pl.BlockSpec((bm, L), lambda i: (i, 0)),
            pl.BlockSpec((1, DIM), lambda i: (0, 0)),
            pl.BlockSpec((DIM, NLAB), lambda i: (0, 0)),
            pl.BlockSpec((1, NLAB), lambda i: (0, 0)),
        ],
        out_specs=pl.BlockSpec((bm, NLAB), lambda i: (i, 0)),
        out_shape=jax.ShapeDtypeStruct((B, NLAB), jnp.float32),
    )(acc, doc, row0, Wd, bd2)


def kernel(doc, table, Wd, bd):
    acc = _sc_segsum(doc.reshape(B * L), table)
    row0 = lax.slice(table, (0, 0), (1, DIM))
    return _tc_decode(acc, doc, row0, Wd, bd.reshape(1, NLAB))


# speedup vs baseline: 1.0812x; 1.0812x over previous
import functools

import jax
import jax.numpy as jnp
from jax import lax
from jax.experimental import pallas as pl
from jax.experimental.pallas import tpu as pltpu
from jax.experimental.pallas import tpu_sc as plsc

VOCAB = 1000000
DIM = 64
B = 4096
L = 200
NLAB = 1000

NC = 2
NS = 16
NW = NC * NS
DOCS_PER_TILE = B // NW
TOK_PER_TILE = DOCS_PER_TILE * L
WDIM = 2 * DIM

CH0 = 128
CH1 = L - CH0


def _sc_segsum(doc_flat, table):
    mesh = plsc.VectorSubcoreMesh(core_axis_name="c", subcore_axis_name="s")

    @functools.partial(
        pl.kernel,
        mesh=mesh,
        out_type=jax.ShapeDtypeStruct((B * DIM,), jnp.float32),
        compiler_params=pltpu.CompilerParams(use_tc_tiling_on_sc=False),
        scratch_types=[
            pltpu.VMEM((TOK_PER_TILE + 16,), jnp.int32),
            pltpu.VMEM((4, L, DIM), jnp.float32),
            pltpu.VMEM((DOCS_PER_TILE * DIM,), jnp.float32),
        ] + [pltpu.SemaphoreType.DMA] * 4,
    )
    def segsum(doc_hbm, tab_hbm, out_hbm, idx_v, rows_v, acc_v, *sems):
        wid = lax.axis_index("s") * NC + lax.axis_index("c")
        base = wid * TOK_PER_TILE

        pltpu.sync_copy(doc_hbm.at[pl.ds(base, TOK_PER_TILE)],
                        idx_v.at[pl.ds(0, TOK_PER_TILE)])

        def gathers(b, buf):
            sem = sems[buf]
            return (
                pltpu.make_async_copy(
                    tab_hbm.at[idx_v.at[pl.ds(b * L, CH0)]],
                    rows_v.at[buf, pl.ds(0, CH0)], sem),
                pltpu.make_async_copy(
                    tab_hbm.at[idx_v.at[pl.ds(b * L + CH0, CH1)]],
                    rows_v.at[buf, pl.ds(CH0, CH1)], sem),
            )

        def issue(b, buf):
            for g in gathers(b, buf):
                g.start()

        def drain(b, buf):
            for g in gathers(b, buf):
                g.wait()

        issue(0, 0)
        issue(1, 1)
        issue(2, 2)

        def per_doc(bb, _):
            for phase in range(4):
                b = 4 * bb + phase
                buf = phase

                @pl.when(b + 3 < DOCS_PER_TILE)
                def _prefetch():
                    issue(b + 3, (phase + 3) % 4)

                drain(b, buf)

                zero = jnp.zeros((16,), jnp.float32)

                def tok(t, accs):
                    new = []
                    for d in range(4):
                        new.append(accs[d] + rows_v[buf, t, pl.ds(16 * d, 16)])
                    return tuple(new)

                accs = lax.fori_loop(0, L, tok, (zero,) * 4)
                for d in range(4):
                    acc_v[pl.ds(b * DIM + 16 * d, 16)] = accs[d]
            return _

        lax.fori_loop(0, DOCS_PER_TILE // 4, per_doc, 0)
        pltpu.sync_copy(
            acc_v,
            out_hbm.at[pl.ds(wid * DOCS_PER_TILE * DIM, DOCS_PER_TILE * DIM)])

    return segsum(doc_flat, table)


def _tc_body(acc_ref, doc_ref, row0_ref, wd_ref, bd_ref, out_ref):
    npad = jnp.sum((doc_ref[...] == 0).astype(jnp.float32), axis=1,
                   keepdims=True)
    cnt = jnp.maximum(float(L) - npad, 1.0)
    enc = (acc_ref[...] - npad * row0_ref[...]) / cnt
    out_ref[...] = jnp.dot(enc, wd_ref[...],
                           preferred_element_type=jnp.float32) + bd_ref[...]


def _tc_decode(acc, doc, row0, Wd, bd2):
    bm = 512
    grid = B // bm
    return pl.pallas_call(
        _tc_body,
        grid=(grid,),
        in_specs=[
            pl.BlockSpec((bm, DIM), lambda i: (i, 0)),
            pl.BlockSpec((bm, L), lambda i: (i, 0)),
            pl.BlockSpec((1, DIM), lambda i: (0, 0)),
            pl.BlockSpec((DIM, NLAB), lambda i: (0, 0)),
            pl.BlockSpec((1, NLAB), lambda i: (0, 0)),
        ],
        out_specs=pl.BlockSpec((bm, NLAB), lambda i: (i, 0)),
        out_shape=jax.ShapeDtypeStruct((B, NLAB), jnp.float32),
    )(acc, doc, row0, Wd, bd2)


def kernel(doc, table, Wd, bd):
    acc_flat = _sc_segsum(doc.reshape(B * L), table)
    acc = acc_flat.reshape(B, DIM)
    row0 = lax.slice(table, (0, 0), (1, DIM))
    return _tc_decode(acc, doc, row0, Wd, bd.reshape(1, NLAB))
